# Initial kernel scaffold; baseline (speedup 1.0000x reference)
import jax
import jax.numpy as jnp
from jax.experimental import pallas as pl


def kernel(seq_indices, token_embed, pos_embed):
    raise NotImplementedError("write your pallas kernel here")


# speedup vs baseline: 11.5556x; 11.5556x over previous
import functools

import jax
import jax.numpy as jnp
from jax.experimental import pallas as pl
from jax.experimental.pallas import tpu as pltpu

_BB = 8
_LL = 512


def _body(idx_ref, tok_ref, pos_ref, out_ref):
    idx = idx_ref[...]
    oh = (idx[..., None] == jax.lax.broadcasted_iota(jnp.int32, (1, 1, 8), 2)
          ).astype(jnp.float32)
    tok_rows = jax.lax.dot_general(
        oh.reshape(_BB * _LL, 8), tok_ref[...],
        (((1,), (0,)), ((), ())), preferred_element_type=jnp.float32)
    out_ref[...] = tok_rows.reshape(_BB, _LL, 64) + pos_ref[...][None]


def kernel(seq_indices, token_embed, pos_embed):
    B, L = seq_indices.shape
    D = token_embed.shape[1]
    tok8 = jnp.zeros((8, D), jnp.float32).at[:5].set(token_embed)
    grid = (L // _LL, B // _BB)
    return pl.pallas_call(
        _body,
        grid=grid,
        in_specs=[
            pl.BlockSpec((_BB, _LL), lambda li, bi: (bi, li)),
            pl.BlockSpec((8, D), lambda li, bi: (0, 0)),
            pl.BlockSpec((_LL, D), lambda li, bi: (li, 0)),
        ],
        out_specs=pl.BlockSpec((_BB, _LL, D), lambda li, bi: (bi, li, 0)),
        out_shape=jax.ShapeDtypeStruct((B, L, D), jnp.float32),
    )(seq_indices, tok8, pos_embed)
